# baseline (speedup 1.0000x reference)
import functools

import jax
import jax.numpy as jnp
import numpy as np
from jax.experimental import pallas as pl
from jax.experimental.pallas import tpu as pltpu
from jax.sharding import Mesh, PartitionSpec as P


def _conv_relu_stats_kernel(x_ref, w_ref, y_ref, stats_ref, *, K, L, R):
    pad = K // 2
    cin = x_ref.shape[1]
    z = jnp.zeros((cin, pad), jnp.bfloat16)

    cols = []
    for r in range(R):
        xp = jnp.concatenate([z, x_ref[r].astype(jnp.bfloat16), z], axis=1)
        cols.append(jnp.concatenate(
            [xp[:, k:k + L] for k in range(K)], axis=0))
    im2col = jnp.concatenate(cols, axis=1)

    acc = jax.lax.dot_general(
        w_ref[...], im2col,
        dimension_numbers=(((1,), (0,)), ((), ())),
        preferred_element_type=jnp.float32)
    acc = jnp.maximum(acc, 0.0)

    for r in range(R):
        y_ref[r] = acc[:, r * L:(r + 1) * L].astype(y_ref.dtype)
    s = jnp.sum(acc, axis=1)
    s2 = jnp.sum(acc * acc, axis=1)
    stats_ref[0] = jnp.stack([s, s2], axis=0)


def _bn_apply_kernel(y_ref, tot_ref, g_ref, b_ref, o_ref, *, count, eps):
    mean = tot_ref[0, 0] / count
    var = tot_ref[0, 1] / count - mean * mean
    inv = jax.lax.rsqrt(var + eps)
    scale = (g_ref[0] * inv)[None, :, None]
    shift = (b_ref[0] - mean * g_ref[0] * inv)[None, :, None]
    y = y_ref[...].astype(jnp.float32)
    o_ref[...] = (y * scale + shift).astype(o_ref.dtype)


def _pick_rows(b):
    for r in (8, 4, 2):
        if b % r == 0:
            return r
    return 1


def _conv_bn_local(x, w, gamma2d, beta2d, count, eps, axis_name):
    Bl, Cin, L = x.shape
    Cout, KC = w.shape
    R = _pick_rows(Bl)
    nG = Bl // R

    conv = functools.partial(_conv_relu_stats_kernel, K=KC // Cin, L=L, R=R)
    y, stats = pl.pallas_call(
        conv,
        out_shape=(
            jax.ShapeDtypeStruct((Bl, Cout, L), jnp.bfloat16),
            jax.ShapeDtypeStruct((nG, 2, Cout), jnp.float32),
        ),
        grid=(nG,),
        in_specs=[
            pl.BlockSpec((R, Cin, L), lambda g: (g, 0, 0)),
            pl.BlockSpec((Cout, KC), lambda g: (0, 0)),
        ],
        out_specs=(
            pl.BlockSpec((R, Cout, L), lambda g: (g, 0, 0)),
            pl.BlockSpec((1, 2, Cout), lambda g: (g, 0, 0)),
        ),
        compiler_params=pltpu.CompilerParams(
            dimension_semantics=("parallel",),
            vmem_limit_bytes=100 * 1024 * 1024),
    )(x, w)

    totals = jnp.sum(stats, axis=0)
    if axis_name is not None:
        totals = jax.lax.psum(totals, axis_name)

    bn = functools.partial(_bn_apply_kernel, count=count, eps=eps)
    out = pl.pallas_call(
        bn,
        out_shape=jax.ShapeDtypeStruct((Bl, Cout, L), x.dtype),
        grid=(nG,),
        in_specs=[
            pl.BlockSpec((R, Cout, L), lambda g: (g, 0, 0)),
            pl.BlockSpec((1, 2, Cout), lambda g: (0, 0, 0)),
            pl.BlockSpec((1, Cout), lambda g: (0, 0)),
            pl.BlockSpec((1, Cout), lambda g: (0, 0)),
        ],
        out_specs=pl.BlockSpec((R, Cout, L), lambda g: (g, 0, 0)),
        compiler_params=pltpu.CompilerParams(
            dimension_semantics=("parallel",),
            vmem_limit_bytes=100 * 1024 * 1024),
    )(y, totals[None], gamma2d, beta2d)
    return out


def kernel(x, weight, gamma, beta, *, eps=1e-5):
    B, Cin, L = x.shape
    Cout, _, K = weight.shape

    w = jnp.transpose(weight, (0, 2, 1)).reshape(Cout, K * Cin).astype(jnp.bfloat16)
    gamma2d = gamma.reshape(1, Cout)
    beta2d = beta.reshape(1, Cout)
    count = float(B * L)

    devs = jax.devices()
    n_dev = 2 if (len(devs) >= 2 and B % 2 == 0) else 1
    if n_dev == 1:
        return _conv_bn_local(x, w, gamma2d, beta2d, count, eps, None)

    mesh = Mesh(np.array(devs[:n_dev]), ("d",))
    f = jax.shard_map(
        functools.partial(_conv_bn_local, count=count, eps=eps, axis_name="d"),
        mesh=mesh,
        in_specs=(P("d"), P(), P(), P()),
        out_specs=P("d"),
        check_vma=False,
    )
    return f(x, w, gamma2d, beta2d)


# speedup vs baseline: 7.0017x; 7.0017x over previous
import functools

import jax
import jax.numpy as jnp
from jax.experimental import pallas as pl
from jax.experimental.pallas import tpu as pltpu


def _fused_kernel(x_ref, w_ref, g_ref, b_ref, o_ref, y_scr, st_scr,
                  *, K, L, R, nG, count, eps):
    step = pl.program_id(0)
    pad = K // 2
    cin = x_ref.shape[1]

    @pl.when(step < nG)
    def _phase0():
        z = jnp.zeros((cin, pad), jnp.bfloat16)
        cols = []
        for r in range(R):
            xp = jnp.concatenate([z, x_ref[r].astype(jnp.bfloat16), z], axis=1)
            cols.append(jnp.concatenate(
                [xp[:, k:k + L] for k in range(K)], axis=0))
        im2col = jnp.concatenate(cols, axis=1)

        acc = jax.lax.dot_general(
            w_ref[...], im2col,
            dimension_numbers=(((1,), (0,)), ((), ())),
            preferred_element_type=jnp.float32)
        acc = jnp.maximum(acc, 0.0)

        for r in range(R):
            y_scr[step * R + r] = acc[:, r * L:(r + 1) * L].astype(y_scr.dtype)

        s = jnp.sum(acc, axis=1)
        s2 = jnp.sum(acc * acc, axis=1)
        part = jnp.stack([s, s2], axis=0)

        @pl.when(step == 0)
        def _init():
            st_scr[...] = part

        @pl.when(step > 0)
        def _accum():
            st_scr[...] = st_scr[...] + part

    @pl.when(step >= nG)
    def _phase1():
        mean = st_scr[0] / count
        var = st_scr[1] / count - mean * mean
        inv = jax.lax.rsqrt(var + eps)
        scale = (g_ref[0] * inv)[None, :, None]
        shift = (b_ref[0] - mean * g_ref[0] * inv)[None, :, None]
        gp = step - nG
        y = y_scr[pl.ds(gp * R, R)].astype(jnp.float32)
        o_ref[...] = (y * scale + shift).astype(o_ref.dtype)


def _pick_rows(b):
    for r in (4, 2):
        if b % r == 0:
            return r
    return 1


def kernel(x, weight, gamma, beta, *, eps=1e-5):
    B, Cin, L = x.shape
    Cout, _, K = weight.shape
    R = _pick_rows(B)
    nG = B // R

    w = jnp.transpose(weight, (0, 2, 1)).reshape(Cout, K * Cin).astype(jnp.bfloat16)

    fused = functools.partial(_fused_kernel, K=K, L=L, R=R, nG=nG,
                              count=float(B * L), eps=eps)
    out = pl.pallas_call(
        fused,
        out_shape=jax.ShapeDtypeStruct((B, Cout, L), x.dtype),
        grid=(2 * nG,),
        in_specs=[
            pl.BlockSpec((R, Cin, L), lambda s: (jnp.minimum(s, nG - 1), 0, 0)),
            pl.BlockSpec((Cout, K * Cin), lambda s: (0, 0)),
            pl.BlockSpec((1, Cout), lambda s: (0, 0)),
            pl.BlockSpec((1, Cout), lambda s: (0, 0)),
        ],
        out_specs=pl.BlockSpec(
            (R, Cout, L), lambda s: (jnp.maximum(s - nG, 0), 0, 0)),
        scratch_shapes=[
            pltpu.VMEM((B, Cout, L), jnp.bfloat16),
            pltpu.VMEM((2, Cout), jnp.float32),
        ],
        compiler_params=pltpu.CompilerParams(
            dimension_semantics=("arbitrary",),
            vmem_limit_bytes=64 * 1024 * 1024),
    )(x, w, gamma.reshape(1, Cout), beta.reshape(1, Cout))
    return out


# speedup vs baseline: 7.0170x; 1.0022x over previous
import functools

import jax
import jax.numpy as jnp
from jax.experimental import pallas as pl
from jax.experimental.pallas import tpu as pltpu


def _fused_kernel(x_ref, w_ref, g_ref, b_ref, o_ref, y_scr, st_scr,
                  *, K, L, R0, R1, nG0, count, eps):
    step = pl.program_id(0)
    pad = K // 2
    cin = x_ref.shape[1]

    @pl.when(step < nG0)
    def _phase0():
        z = jnp.zeros((cin, pad), jnp.bfloat16)
        cols = []
        for r in range(R0):
            xp = jnp.concatenate([z, x_ref[r].astype(jnp.bfloat16), z], axis=1)
            cols.append(jnp.concatenate(
                [xp[:, k:k + L] for k in range(K)], axis=0))
        im2col = jnp.concatenate(cols, axis=1)

        acc = jax.lax.dot_general(
            w_ref[...], im2col,
            dimension_numbers=(((1,), (0,)), ((), ())),
            preferred_element_type=jnp.float32)
        acc = jnp.maximum(acc, 0.0)

        for r in range(R0):
            y_scr[step * R0 + r] = acc[:, r * L:(r + 1) * L].astype(y_scr.dtype)

        s = jnp.sum(acc, axis=1)
        s2 = jnp.sum(acc * acc, axis=1)
        part = jnp.stack([s, s2], axis=0)

        @pl.when(step == 0)
        def _init():
            st_scr[...] = part

        @pl.when(step > 0)
        def _accum():
            st_scr[...] = st_scr[...] + part

    @pl.when(step >= nG0)
    def _phase1():
        mean = st_scr[0] / count
        var = st_scr[1] / count - mean * mean
        inv = jax.lax.rsqrt(var + eps)
        scale = (g_ref[0] * inv)[None, :, None]
        shift = (b_ref[0] - mean * g_ref[0] * inv)[None, :, None]
        gp = step - nG0
        y = y_scr[pl.ds(gp * R1, R1)].astype(jnp.float32)
        o_ref[...] = (y * scale + shift).astype(o_ref.dtype)


def _pick_rows(b, cands):
    for r in cands:
        if b % r == 0:
            return r
    return 1


def kernel(x, weight, gamma, beta, *, eps=1e-5):
    B, Cin, L = x.shape
    Cout, _, K = weight.shape
    R0 = _pick_rows(B, (4, 2))
    R1 = _pick_rows(B, (8, 4, 2))
    nG0 = B // R0
    nG1 = B // R1

    w = jnp.transpose(weight, (0, 2, 1)).reshape(Cout, K * Cin).astype(jnp.bfloat16)

    fused = functools.partial(_fused_kernel, K=K, L=L, R0=R0, R1=R1, nG0=nG0,
                              count=float(B * L), eps=eps)
    out = pl.pallas_call(
        fused,
        out_shape=jax.ShapeDtypeStruct((B, Cout, L), x.dtype),
        grid=(nG0 + nG1,),
        in_specs=[
            pl.BlockSpec((R0, Cin, L), lambda s: (jnp.minimum(s, nG0 - 1), 0, 0)),
            pl.BlockSpec((Cout, K * Cin), lambda s: (0, 0)),
            pl.BlockSpec((1, Cout), lambda s: (0, 0)),
            pl.BlockSpec((1, Cout), lambda s: (0, 0)),
        ],
        out_specs=pl.BlockSpec(
            (R1, Cout, L), lambda s: (jnp.maximum(s - nG0, 0), 0, 0)),
        scratch_shapes=[
            pltpu.VMEM((B, Cout, L), jnp.bfloat16),
            pltpu.VMEM((2, Cout), jnp.float32),
        ],
        compiler_params=pltpu.CompilerParams(
            dimension_semantics=("arbitrary",),
            vmem_limit_bytes=64 * 1024 * 1024),
    )(x, w, gamma.reshape(1, Cout), beta.reshape(1, Cout))
    return out


# speedup vs baseline: 7.3289x; 1.0445x over previous
import functools

import jax
import jax.numpy as jnp
from jax.experimental import pallas as pl
from jax.experimental.pallas import tpu as pltpu


def _fused_kernel(x_ref, w_ref, g_ref, b_ref, o_ref, y_scr, st_scr,
                  *, K, L, R0, R1, nG0, count, eps):
    step = pl.program_id(0)
    pad = K // 2
    cin = x_ref.shape[1]

    @pl.when(step < nG0)
    def _phase0():
        z = jnp.zeros((cin, pad), jnp.bfloat16)
        cols = []
        for r in range(R0):
            xp = jnp.concatenate([z, x_ref[r].astype(jnp.bfloat16), z], axis=1)
            cols.append(jnp.concatenate(
                [xp[:, k:k + L] for k in range(K)], axis=0))
        im2col = jnp.concatenate(cols, axis=1)

        acc = jax.lax.dot_general(
            w_ref[...], im2col,
            dimension_numbers=(((1,), (0,)), ((), ())),
            preferred_element_type=jnp.float32)
        acc = jnp.maximum(acc, 0.0)

        for r in range(R0):
            y_scr[step * R0 + r] = acc[:, r * L:(r + 1) * L].astype(y_scr.dtype)

        s = jnp.sum(acc, axis=1)
        s2 = jnp.sum(acc * acc, axis=1)
        part = jnp.stack([s, s2], axis=0)

        @pl.when(step == 0)
        def _init():
            st_scr[...] = part

        @pl.when(step > 0)
        def _accum():
            st_scr[...] = st_scr[...] + part

    @pl.when(step >= nG0)
    def _phase1():
        mean = st_scr[0] / count
        var = st_scr[1] / count - mean * mean
        inv = jax.lax.rsqrt(var + eps)
        scale = (g_ref[0] * inv)[None, :, None]
        shift = (b_ref[0] - mean * g_ref[0] * inv)[None, :, None]
        gp = step - nG0
        y = y_scr[pl.ds(gp * R1, R1)].astype(jnp.float32)
        o_ref[...] = (y * scale + shift).astype(o_ref.dtype)


def _pick_rows(b, cands):
    for r in cands:
        if b % r == 0:
            return r
    return 1


def kernel(x, weight, gamma, beta, *, eps=1e-5):
    B, Cin, L = x.shape
    Cout, _, K = weight.shape
    R0 = _pick_rows(B, (8, 4, 2))
    R1 = _pick_rows(B, (4, 2))
    nG0 = B // R0
    nG1 = B // R1

    w = jnp.transpose(weight, (0, 2, 1)).reshape(Cout, K * Cin).astype(jnp.bfloat16)

    fused = functools.partial(_fused_kernel, K=K, L=L, R0=R0, R1=R1, nG0=nG0,
                              count=float(B * L), eps=eps)
    out = pl.pallas_call(
        fused,
        out_shape=jax.ShapeDtypeStruct((B, Cout, L), x.dtype),
        grid=(nG0 + nG1,),
        in_specs=[
            pl.BlockSpec((R0, Cin, L), lambda s: (jnp.minimum(s, nG0 - 1), 0, 0)),
            pl.BlockSpec((Cout, K * Cin), lambda s: (0, 0)),
            pl.BlockSpec((1, Cout), lambda s: (0, 0)),
            pl.BlockSpec((1, Cout), lambda s: (0, 0)),
        ],
        out_specs=pl.BlockSpec(
            (R1, Cout, L), lambda s: (jnp.maximum(s - nG0, 0), 0, 0)),
        scratch_shapes=[
            pltpu.VMEM((B, Cout, L), jnp.bfloat16),
            pltpu.VMEM((2, Cout), jnp.float32),
        ],
        compiler_params=pltpu.CompilerParams(
            dimension_semantics=("arbitrary",),
            vmem_limit_bytes=64 * 1024 * 1024),
    )(x, w, gamma.reshape(1, Cout), beta.reshape(1, Cout))
    return out


# speedup vs baseline: 7.3511x; 1.0030x over previous
import functools

import jax
import jax.numpy as jnp
from jax.experimental import pallas as pl
from jax.experimental.pallas import tpu as pltpu


def _fused_kernel(x_ref, w_ref, g_ref, b_ref, o_ref, y_scr, st_scr,
                  *, K, L, R0, R1, nG0, count, eps):
    step = pl.program_id(0)
    pad = K // 2
    cin = x_ref.shape[1]

    @pl.when(step < nG0)
    def _phase0():
        z = jnp.zeros((cin, pad), jnp.bfloat16)
        H = 2 if R0 % 2 == 0 else 1
        RH = R0 // H
        s_parts, s2_parts = [], []
        for h in range(H):
            cols = []
            for r in range(h * RH, (h + 1) * RH):
                xp = jnp.concatenate(
                    [z, x_ref[r].astype(jnp.bfloat16), z], axis=1)
                cols.append(jnp.concatenate(
                    [xp[:, k:k + L] for k in range(K)], axis=0))
            im2col = jnp.concatenate(cols, axis=1)

            acc = jax.lax.dot_general(
                w_ref[...], im2col,
                dimension_numbers=(((1,), (0,)), ((), ())),
                preferred_element_type=jnp.float32)
            acc = jnp.maximum(acc, 0.0)

            for i in range(RH):
                y_scr[step * R0 + h * RH + i] = (
                    acc[:, i * L:(i + 1) * L].astype(y_scr.dtype))
            s_parts.append(jnp.sum(acc, axis=1))
            s2_parts.append(jnp.sum(acc * acc, axis=1))

        part = jnp.stack([sum(s_parts), sum(s2_parts)], axis=0)

        @pl.when(step == 0)
        def _init():
            st_scr[...] = part

        @pl.when(step > 0)
        def _accum():
            st_scr[...] = st_scr[...] + part

    @pl.when(step >= nG0)
    def _phase1():
        mean = st_scr[0] / count
        var = st_scr[1] / count - mean * mean
        inv = jax.lax.rsqrt(var + eps)
        scale = (g_ref[0] * inv)[None, :, None]
        shift = (b_ref[0] - mean * g_ref[0] * inv)[None, :, None]
        gp = step - nG0
        y = y_scr[pl.ds(gp * R1, R1)].astype(jnp.float32)
        o_ref[...] = (y * scale + shift).astype(o_ref.dtype)


def _pick_rows(b, cands):
    for r in cands:
        if b % r == 0:
            return r
    return 1


def kernel(x, weight, gamma, beta, *, eps=1e-5):
    B, Cin, L = x.shape
    Cout, _, K = weight.shape
    R0 = _pick_rows(B, (8, 4, 2))
    R1 = _pick_rows(B, (4, 2))
    nG0 = B // R0
    nG1 = B // R1

    w = jnp.transpose(weight, (0, 2, 1)).reshape(Cout, K * Cin).astype(jnp.bfloat16)

    fused = functools.partial(_fused_kernel, K=K, L=L, R0=R0, R1=R1, nG0=nG0,
                              count=float(B * L), eps=eps)
    out = pl.pallas_call(
        fused,
        out_shape=jax.ShapeDtypeStruct((B, Cout, L), x.dtype),
        grid=(nG0 + nG1,),
        in_specs=[
            pl.BlockSpec((R0, Cin, L), lambda s: (jnp.minimum(s, nG0 - 1), 0, 0)),
            pl.BlockSpec((Cout, K * Cin), lambda s: (0, 0)),
            pl.BlockSpec((1, Cout), lambda s: (0, 0)),
            pl.BlockSpec((1, Cout), lambda s: (0, 0)),
        ],
        out_specs=pl.BlockSpec(
            (R1, Cout, L), lambda s: (jnp.maximum(s - nG0, 0), 0, 0)),
        scratch_shapes=[
            pltpu.VMEM((B, Cout, L), jnp.bfloat16),
            pltpu.VMEM((2, Cout), jnp.float32),
        ],
        compiler_params=pltpu.CompilerParams(
            dimension_semantics=("arbitrary",),
            vmem_limit_bytes=64 * 1024 * 1024),
    )(x, w, gamma.reshape(1, Cout), beta.reshape(1, Cout))
    return out


# speedup vs baseline: 7.3683x; 1.0023x over previous
import functools

import jax
import jax.numpy as jnp
from jax.experimental import pallas as pl
from jax.experimental.pallas import tpu as pltpu


def _fused_kernel(x_ref, w_ref, g_ref, b_ref, o_ref, y_scr, st_scr,
                  *, K, L, R0, R1, nG0, count, eps):
    step = pl.program_id(0)
    pad = K // 2
    cin = x_ref.shape[1]

    @pl.when(step < nG0)
    def _phase0():
        z = jnp.zeros((cin, pad), jnp.bfloat16)
        H = 2 if R0 % 2 == 0 else 1
        RH = R0 // H
        s_parts, s2_parts = [], []
        for h in range(H):
            cols = []
            for r in range(h * RH, (h + 1) * RH):
                xp = jnp.concatenate(
                    [z, x_ref[r].astype(jnp.bfloat16), z], axis=1)
                cols.append(jnp.concatenate(
                    [xp[:, k:k + L] for k in range(K)], axis=0))
            im2col = jnp.concatenate(cols, axis=1)

            acc = jax.lax.dot_general(
                w_ref[...], im2col,
                dimension_numbers=(((1,), (0,)), ((), ())),
                preferred_element_type=jnp.float32)
            acc = jnp.maximum(acc, 0.0)

            for i in range(RH):
                y_scr[step * R0 + h * RH + i] = (
                    acc[:, i * L:(i + 1) * L].astype(y_scr.dtype))
            s_parts.append(jnp.sum(acc, axis=1))
            s2_parts.append(jnp.sum(acc * acc, axis=1))

        part = jnp.stack([sum(s_parts), sum(s2_parts)], axis=0)

        @pl.when(step == 0)
        def _init():
            st_scr[...] = part

        @pl.when(step > 0)
        def _accum():
            st_scr[...] = st_scr[...] + part

    @pl.when(step >= nG0)
    def _phase1():
        gp = step - nG0

        @pl.when(gp == 0)
        def _fold():
            mean = st_scr[0] / count
            var = st_scr[1] / count - mean * mean
            inv = jax.lax.rsqrt(var + eps)
            scale = g_ref[0] * inv
            st_scr[...] = jnp.stack([scale, b_ref[0] - mean * scale], axis=0)

        scale = st_scr[0][None, :, None]
        shift = st_scr[1][None, :, None]
        y = y_scr[pl.ds(gp * R1, R1)].astype(jnp.float32)
        o_ref[...] = (y * scale + shift).astype(o_ref.dtype)


def _pick_rows(b, cands):
    for r in cands:
        if b % r == 0:
            return r
    return 1


def kernel(x, weight, gamma, beta, *, eps=1e-5):
    B, Cin, L = x.shape
    Cout, _, K = weight.shape
    R0 = _pick_rows(B, (8, 4, 2))
    R1 = _pick_rows(B, (4, 2))
    nG0 = B // R0
    nG1 = B // R1

    w = jnp.transpose(weight, (0, 2, 1)).reshape(Cout, K * Cin).astype(jnp.bfloat16)

    fused = functools.partial(_fused_kernel, K=K, L=L, R0=R0, R1=R1, nG0=nG0,
                              count=float(B * L), eps=eps)
    out = pl.pallas_call(
        fused,
        out_shape=jax.ShapeDtypeStruct((B, Cout, L), x.dtype),
        grid=(nG0 + nG1,),
        in_specs=[
            pl.BlockSpec((R0, Cin, L), lambda s: (jnp.minimum(s, nG0 - 1), 0, 0)),
            pl.BlockSpec((Cout, K * Cin), lambda s: (0, 0)),
            pl.BlockSpec((1, Cout), lambda s: (0, 0)),
            pl.BlockSpec((1, Cout), lambda s: (0, 0)),
        ],
        out_specs=pl.BlockSpec(
            (R1, Cout, L), lambda s: (jnp.maximum(s - nG0, 0), 0, 0)),
        scratch_shapes=[
            pltpu.VMEM((B, Cout, L), jnp.bfloat16),
            pltpu.VMEM((2, Cout), jnp.float32),
        ],
        compiler_params=pltpu.CompilerParams(
            dimension_semantics=("arbitrary",),
            vmem_limit_bytes=64 * 1024 * 1024),
    )(x, w, gamma.reshape(1, Cout), beta.reshape(1, Cout))
    return out
